# baseline (speedup 1.0000x reference)
import functools

import jax
import jax.numpy as jnp
from jax import lax
from jax.experimental import pallas as pl
from jax.experimental.pallas import tpu as pltpu
from jax.experimental.pallas import tpu_sc as plsc

C = 32
CP = C // 2
W = 512
HW = W * W
NPLANES = 3
NPTS = 524288
NTAP = 12

_info = plsc.get_sparse_core_info()
NC, NS, L = _info.num_cores, _info.num_subcores, _info.num_lanes
NW = NC * NS
PPW = NPTS // NW
B = 128
NB = PPW // B
G = B // L


@functools.partial(
    pl.kernel,
    mesh=plsc.VectorSubcoreMesh(core_axis_name="c", subcore_axis_name="s"),
    out_type=jax.ShapeDtypeStruct((NPTS * C,), jnp.float32),
    compiler_params=pltpu.CompilerParams(
        use_tc_tiling_on_sc=False, needs_layout_passes=False),
    scratch_types=[
        pltpu.VMEM((B, NPLANES), jnp.float32),
        pltpu.VMEM((2, NTAP, B), jnp.int32),
        pltpu.VMEM((2, NTAP, B), jnp.int32),
        pltpu.VMEM((2, NTAP, B, CP), jnp.int32),
        pltpu.VMEM((B * C,), jnp.float32),
        pltpu.SemaphoreType.DMA,
        pltpu.SemaphoreType.DMA,
    ],
)
def _tri_gather(xyz_hbm, t0_hbm, t1_hbm, t2_hbm, out_hbm,
                uvw_v, idx_v, w_v, rows_v, out_v, sem0, sem1):
    wid = lax.axis_index("s") * NC + lax.axis_index("c")
    base = wid * PPW
    iota = lax.iota(jnp.int32, L)
    tabs = (t0_hbm, t1_hbm, t2_hbm)

    def build(j, d):
        pltpu.sync_copy(xyz_hbm.at[pl.ds(base + j * B, B)], uvw_v)

        def g_body(g, c0):
            lp = g * L + iota
            slb = pl.ds(g * L, L)
            coords = tuple(
                plsc.load_gather(uvw_v, [lp, jnp.full((L,), k, jnp.int32)])
                for k in range(NPLANES))
            for p in range(NPLANES):
                u = coords[p]
                v = coords[(p + 1) % NPLANES]
                iu = u * (W * 0.5) + (W - 1) * 0.5
                iv = v * (W * 0.5) + (W - 1) * 0.5
                iu0 = iu.astype(jnp.int32)
                iv0 = iv.astype(jnp.int32)
                fu = iu - iu0.astype(jnp.float32)
                fv = iv - iv0.astype(jnp.float32)
                u1ok = iu0 < (W - 1)
                v1ok = iv0 < (W - 1)
                iu1 = jnp.where(u1ok, iu0 + 1, W - 1)
                iv1 = jnp.where(v1ok, iv0 + 1, W - 1)
                wu1 = jnp.where(u1ok, fu, 0.0)
                wv1 = jnp.where(v1ok, fv, 0.0)
                wu0 = 1.0 - fu
                wv0 = 1.0 - fv
                r0 = iv0 * W
                r1 = iv1 * W
                idx_v[d, 4 * p + 0, slb] = r0 + iu0
                idx_v[d, 4 * p + 1, slb] = r0 + iu1
                idx_v[d, 4 * p + 2, slb] = r1 + iu0
                idx_v[d, 4 * p + 3, slb] = r1 + iu1
                wts = (wu0 * wv0, wu1 * wv0, wu0 * wv1, wu1 * wv1)
                for t in range(4):
                    w_v[d, 4 * p + t, slb] = plsc.bitcast(
                        plsc.pack(wts[t], wts[t],
                                  format=plsc.PackFormat.INTERLEAVED),
                        jnp.int32)
            return c0

        lax.fori_loop(0, G, g_body, 0)

    def fire(d):
        sem = sem0 if d == 0 else sem1
        for t in range(NTAP):
            pltpu.async_copy(
                tabs[t // 4].at[idx_v.at[d, t]], rows_v.at[d, t], sem)

    def drain(d):
        sem = sem0 if d == 0 else sem1
        for t in range(NTAP):
            pltpu.make_async_copy(
                tabs[t // 4].at[idx_v.at[d, t]], rows_v.at[d, t], sem).wait()

    def accumulate(j, d):
        def group_body(g, c2):
            base_pp = g * L
            wwords = [w_v[d, t, pl.ds(base_pp, L)] for t in range(NTAP)]
            for lane in range(L):
                pp = base_pp + lane
                acc_a = None
                acc_b = None
                for p in range(NPLANES):
                    pacc = None
                    for t in range(4 * p, 4 * p + 4):
                        wsplat = plsc.bitcast(
                            jnp.full((L,), wwords[t][lane], jnp.int32),
                            jnp.bfloat16)
                        row = plsc.bitcast(rows_v[d, t, pp, :], jnp.bfloat16)
                        term = wsplat * row
                        pacc = term if pacc is None else pacc + term
                    a, b = plsc.unpack(
                        pacc, format=plsc.PackFormat.INTERLEAVED,
                        preferred_element_type=jnp.float32)
                    acc_a = a if acc_a is None else acc_a + a
                    acc_b = b if acc_b is None else acc_b + b
                out_v[pl.ds(pp * C, L)] = acc_a
                out_v[pl.ds(pp * C + L, L)] = acc_b
            return c2

        lax.fori_loop(0, G, group_body, 0)
        pltpu.sync_copy(out_v, out_hbm.at[pl.ds((base + j * B) * C, B * C)])

    build(0, 0)
    fire(0)

    def body2(k, carry):
        j0 = 2 * k
        build(j0 + 1, 1)
        fire(1)
        drain(0)
        accumulate(j0, 0)
        jn = jnp.minimum(j0 + 2, NB - 1)
        build(jn, 0)
        fire(0)
        drain(1)
        accumulate(j0 + 1, 1)
        return carry

    lax.fori_loop(0, NB // 2, body2, 0)
    drain(0)


def kernel(xyz, T_xy, T_yz, T_zx):
    def prep(T):
        t = jnp.transpose(T[0], (1, 2, 0)).astype(jnp.bfloat16)
        pairs = jnp.stack([t[..., :CP], t[..., CP:]], axis=-1)
        return lax.bitcast_convert_type(pairs, jnp.int32).reshape(HW, CP)

    flat = _tri_gather(xyz, prep(T_xy), prep(T_yz), prep(T_zx))
    return flat.reshape(NPTS, C)
